# baseline (speedup 1.0000x reference)
import jax
import jax.numpy as jnp
from jax.experimental import pallas as pl
from jax.experimental.pallas import tpu as pltpu

_A_O = 0.1
_K_O = 0.1
_A_ON = 0.006
_K_ON = 0.006
_A_R = 0.1
_K_R = 0.1
_A_RN = 0.006
_K_RN = 0.006
_MAXN = 16

_BIG_I = 2 ** 30
_INF = float("inf")


def _rayleigh(dx, k, a):
    return k / a * dx * jnp.exp(-0.5 * (dx / a) ** 2)


def _negexp(dx, k, a):
    return k * jnp.exp(-a * dx ** 2)


def _pack(x):
    m = x.shape[1]
    return x.reshape(m // 128, 128) if m % 128 == 0 else x


def _packed_iota(shape):
    return (jax.lax.broadcasted_iota(jnp.int32, shape, 0) * shape[1]
            + jax.lax.broadcasted_iota(jnp.int32, shape, 1))


def _topk_mask_full(score, k):
    sel = jnp.zeros(score.shape, jnp.bool_)
    work = score
    for _ in range(k):
        m = jnp.min(work)
        hit = work == m
        sel = jnp.logical_or(sel, hit)
        work = jnp.where(hit, _INF, work)
    return sel


def _topk_mask_rows(score, k):
    sel = jnp.zeros(score.shape, jnp.bool_)
    work = score
    for _ in range(k):
        m = jnp.min(work, axis=1, keepdims=True)
        hit = work == m
        sel = jnp.logical_or(sel, hit)
        work = jnp.where(hit, _INF, work)
    return sel


def _sum_k_smallest_rows(score, k):
    work = score
    acc = jnp.zeros((score.shape[0], 1), jnp.float32)
    for _ in range(k):
        m = jnp.min(work, axis=1, keepdims=True)
        work = jnp.where(work == m, _INF, work)
        acc = acc + m
    return jnp.sum(acc)


def _ng_kernel(data_ref, cat_ref, attract_ref, repel_ref, epochs_ref,
               best_ref, v_ref, adj_ref):
    n, d = data_ref.shape
    na = attract_ref.shape[0]
    nr = repel_ref.shape[0]

    v_ref[...] = data_ref[...]
    best_ref[...] = data_ref[...]
    adj_ref[...] = jnp.zeros((n, n), jnp.float32)

    sq_cat = jnp.sum(cat_ref[...] ** 2, axis=0, keepdims=True)
    sq_d = sq_cat[:, :n]
    sq_a = sq_cat[:, n:n + na]
    sq_r = sq_cat[:, n + na:]

    iota_d = _packed_iota(_pack(sq_d).shape)
    iota_a = _packed_iota(_pack(sq_a).shape)
    lane_n = jax.lax.broadcasted_iota(jnp.int32, (1, n), 1)

    def node_step(s1, carry):
        v1 = v_ref[pl.ds(s1, 1), :]
        sq_v1 = jnp.sum(v1 * v1)
        prod = jnp.dot(v1, cat_ref[...],
                       preferred_element_type=jnp.float32)

        work = _pack(sq_d - 2.0 * prod[:, :n])
        nbr = []
        for j in range(6):
            m = jnp.min(work)
            hit = work == m
            if j > 0:
                nbr.append(jnp.min(jnp.where(hit, iota_d, _BIG_I)))
            work = jnp.where(hit, _INF, work)

        row = adj_ref[pl.ds(s1, 1), :]
        mask5 = (lane_n == nbr[0])
        for idx in nbr[1:]:
            mask5 = jnp.logical_or(mask5, lane_n == idx)
        rowv = jnp.where(mask5, 1.0, row)
        adj_ref[pl.ds(s1, 1), :] = rowv
        for idx in nbr:
            r2 = adj_ref[pl.ds(idx, 1), :]
            adj_ref[pl.ds(idx, 1), :] = jnp.where(lane_n == s1, 1.0, r2)

        work = _pack(sq_a - 2.0 * prod[:, n:n + na])
        gks = []
        for j in range(3):
            m = jnp.min(work)
            hit = work == m
            idx = jnp.min(jnp.where(hit, iota_a, _BIG_I))
            work = jnp.where(hit, _INF, work)
            gks.append(attract_ref[pl.ds(idx, 1), :])

        move = jnp.zeros((1, d), jnp.float32)
        for g in gks:
            diff = g - v1
            dx = jnp.sum(diff * diff)
            move = move + _rayleigh(dx, _A_O, _K_O) * diff

        dxr_p = _pack(sq_r + sq_v1 - 2.0 * prod[:, n + na:])
        sel_r = _topk_mask_full(dxr_p, 10)
        w_p = jnp.where(sel_r, _negexp(dxr_p, _A_R, _K_R), 0.0)
        w_r = w_p.reshape(1, nr)
        move = move - (jnp.dot(w_r, repel_ref[...],
                               preferred_element_type=jnp.float32)
                       - jnp.sum(w_p) * v1)
        v_ref[pl.ds(s1, 1), :] = v1 + move

        keys = _pack(jnp.where(rowv > 0.0, lane_n, lane_n + n))
        nidx = []
        valid = []
        workn = keys
        for j in range(_MAXN):
            m = jnp.min(workn)
            workn = jnp.where(workn == m, _BIG_I, workn)
            valid.append(m < n)
            nidx.append(jnp.where(m < n, m, m - n))

        vk = jnp.concatenate([v_ref[pl.ds(i, 1), :] for i in nidx], axis=0)
        sq_vk = jnp.sum(vk * vk, axis=1, keepdims=True)

        mk = jnp.zeros((_MAXN, d), jnp.float32)
        for g in gks:
            diffn = g - vk
            dxn = jnp.sum(diffn * diffn, axis=1, keepdims=True)
            mk = mk + _rayleigh(dxn, _A_ON, _K_ON) * diffn

        rt = cat_ref[:, n + na:]
        drn = sq_vk + sq_r - 2.0 * jnp.dot(vk, rt,
                                           preferred_element_type=jnp.float32)
        sel_n = _topk_mask_rows(drn, 10)
        w_n = jnp.where(sel_n, _negexp(drn, _A_RN, _K_RN), 0.0)
        w_sum = jnp.sum(w_n, axis=1, keepdims=True)
        mk = mk - (jnp.dot(w_n, repel_ref[...],
                           preferred_element_type=jnp.float32) - w_sum * vk)

        for j in range(_MAXN):
            row_new = jnp.where(valid[j], vk[j:j + 1, :] + mk[j:j + 1, :],
                                vk[j:j + 1, :])
            v_ref[pl.ds(nidx[j], 1), :] = row_new
        return carry

    def epoch_body(e, prev):
        jax.lax.fori_loop(0, n, node_step, 0)
        vv = v_ref[...]
        sq_v = jnp.sum(vv * vv, axis=1, keepdims=True)
        at = cat_ref[:, n:n + na]
        dall = sq_v + sq_a - 2.0 * jnp.dot(vv, at,
                                           preferred_element_type=jnp.float32)
        cur = _sum_k_smallest_rows(dall, 20)
        improved = jnp.logical_not(prev < cur)
        best_ref[...] = jnp.where(improved, v_ref[...], best_ref[...])
        return jnp.where(improved, cur, prev)

    jax.lax.fori_loop(0, epochs_ref[0], epoch_body, _INF)


def kernel(data, attract, repel, epochs):
    n, d = data.shape
    epochs_arr = jnp.asarray(epochs, jnp.int32).reshape(1)
    cat = jnp.concatenate([data.T, attract.T, repel.T], axis=1)
    return pl.pallas_call(
        _ng_kernel,
        out_shape=jax.ShapeDtypeStruct((n, d), jnp.float32),
        in_specs=[
            pl.BlockSpec(memory_space=pltpu.VMEM),
            pl.BlockSpec(memory_space=pltpu.VMEM),
            pl.BlockSpec(memory_space=pltpu.VMEM),
            pl.BlockSpec(memory_space=pltpu.VMEM),
            pl.BlockSpec(memory_space=pltpu.SMEM),
        ],
        out_specs=pl.BlockSpec(memory_space=pltpu.VMEM),
        scratch_shapes=[
            pltpu.VMEM((n, d), jnp.float32),
            pltpu.VMEM((n, n), jnp.float32),
        ],
    )(data, cat, attract, repel, epochs_arr)


# speedup vs baseline: 1.4447x; 1.4447x over previous
import jax
import jax.numpy as jnp
from jax.experimental import pallas as pl
from jax.experimental.pallas import tpu as pltpu

_A_O = 0.1
_K_O = 0.1
_A_ON = 0.006
_K_ON = 0.006
_A_R = 0.1
_K_R = 0.1
_A_RN = 0.006
_K_RN = 0.006
_MAXN = 16

_BIG_I = 2 ** 30
_INF = float("inf")


def _rayleigh(dx, k, a):
    return k / a * dx * jnp.exp(-0.5 * (dx / a) ** 2)


def _negexp(dx, k, a):
    return k * jnp.exp(-a * dx ** 2)


def _mask_k_smallest(score, k, rowwise=False):
    work = score
    for _ in range(k):
        m = jnp.min(work, axis=1, keepdims=True) if rowwise else jnp.min(work)
        work = jnp.where(work == m, _INF, work)
    return work != score


def _cumsum_lanes(x):
    lane = jax.lax.broadcasted_iota(jnp.int32, x.shape, 1)
    y = x
    shift = 1
    while shift < x.shape[1]:
        y = y + jnp.where(lane >= shift, jnp.roll(y, shift, axis=1), 0.0)
        shift *= 2
    return y


def _sum_k_smallest_rows(score, k):
    work = score
    acc = jnp.zeros((score.shape[0], 1), jnp.float32)
    for _ in range(k):
        m = jnp.min(work, axis=1, keepdims=True)
        work = jnp.where(work == m, _INF, work)
        acc = acc + m
    return jnp.sum(acc)


def _ng_kernel(data_ref, cat_ref, attract_ref, repel_ref, epochs_ref,
               best_ref, v_ref, adj_ref):
    n, d = data_ref.shape
    na = attract_ref.shape[0]
    nr = repel_ref.shape[0]

    v_ref[...] = data_ref[...]
    best_ref[...] = data_ref[...]
    adj_ref[...] = jnp.zeros((n, n), jnp.float32)

    sq_cat = jnp.sum(cat_ref[...] ** 2, axis=0, keepdims=True)
    sq_d = sq_cat[:, :n]
    sq_a = sq_cat[:, n:n + na]
    sq_r = sq_cat[:, n + na:]

    lane_n = jax.lax.broadcasted_iota(jnp.int32, (1, n), 1)
    lane_a = jax.lax.broadcasted_iota(jnp.int32, (1, na), 1)
    slot_iota = jax.lax.broadcasted_iota(
        jnp.int32, (_MAXN, 1), 0).astype(jnp.float32)

    def node_step(s1, carry):
        v1 = v_ref[pl.ds(s1, 1), :]
        sq_v1 = jnp.sum(v1 * v1)
        prod = jnp.dot(v1, cat_ref[...],
                       preferred_element_type=jnp.float32)

        ds_sc = sq_d - 2.0 * prod[:, :n]
        work = ds_sc
        nbr = []
        hit0 = None
        for j in range(6):
            m = jnp.min(work)
            if j == 0:
                hit0 = ds_sc == m
            else:
                nbr.append(jnp.min(jnp.where(work == m, lane_n, _BIG_I)))
            work = jnp.where(work == m, _INF, work)
        mask5 = jnp.logical_and(work != ds_sc, jnp.logical_not(hit0))

        row = adj_ref[pl.ds(s1, 1), :]
        rowv = jnp.where(mask5, 1.0, row)
        adj_ref[pl.ds(s1, 1), :] = rowv
        for idx in nbr:
            r2 = adj_ref[pl.ds(idx, 1), :]
            adj_ref[pl.ds(idx, 1), :] = jnp.where(lane_n == s1, 1.0, r2)

        da = sq_a - 2.0 * prod[:, n:n + na]
        work = da
        gidx = []
        for j in range(3):
            m = jnp.min(work)
            gidx.append(jnp.min(jnp.where(work == m, lane_a, _BIG_I)))
            work = jnp.where(work == m, _INF, work)
        sel_a = work != da
        w_a = jnp.where(sel_a, _rayleigh(da + sq_v1, _A_O, _K_O), 0.0)
        move = (jnp.dot(w_a, attract_ref[...],
                        preferred_element_type=jnp.float32)
                - jnp.sum(w_a) * v1)
        gks = [attract_ref[pl.ds(i, 1), :] for i in gidx]

        dxr = sq_r + sq_v1 - 2.0 * prod[:, n + na:]
        sel_r = _mask_k_smallest(dxr, 10)
        w_r = jnp.where(sel_r, _negexp(dxr, _A_R, _K_R), 0.0)
        move = move - (jnp.dot(w_r, repel_ref[...],
                               preferred_element_type=jnp.float32)
                       - jnp.sum(w_r) * v1)
        v_ref[pl.ds(s1, 1), :] = v1 + move

        cnt = jnp.sum(rowv)
        p_set = _cumsum_lanes(rowv)
        p_unset = _cumsum_lanes(1.0 - rowv)
        slotid = jnp.where(rowv > 0.0, p_set - 1.0, cnt + p_unset - 1.0)
        onehot = jnp.where(slotid == slot_iota, 1.0, 0.0)
        valid = slot_iota < cnt

        vk = jnp.dot(onehot, v_ref[...],
                     preferred_element_type=jnp.float32)
        sq_vk = jnp.sum(vk * vk, axis=1, keepdims=True)

        mk = jnp.zeros((_MAXN, d), jnp.float32)
        for g in gks:
            diffn = g - vk
            dxn = jnp.sum(diffn * diffn, axis=1, keepdims=True)
            mk = mk + _rayleigh(dxn, _A_ON, _K_ON) * diffn

        rt = cat_ref[:, n + na:]
        drn = sq_vk + sq_r - 2.0 * jnp.dot(vk, rt,
                                           preferred_element_type=jnp.float32)
        sel_n = _mask_k_smallest(drn, 10, rowwise=True)
        w_n = jnp.where(sel_n, _negexp(drn, _A_RN, _K_RN), 0.0)
        w_sum = jnp.sum(w_n, axis=1, keepdims=True)
        mk = mk - (jnp.dot(w_n, repel_ref[...],
                           preferred_element_type=jnp.float32) - w_sum * vk)

        delta = jnp.where(valid, mk, 0.0)
        v_ref[...] = v_ref[...] + jax.lax.dot_general(
            onehot, delta, (((0,), (0,)), ((), ())),
            preferred_element_type=jnp.float32)
        return carry

    def epoch_body(e, prev):
        jax.lax.fori_loop(0, n, node_step, 0)
        vv = v_ref[...]
        sq_v = jnp.sum(vv * vv, axis=1, keepdims=True)
        at = cat_ref[:, n:n + na]
        dall = sq_v + sq_a - 2.0 * jnp.dot(vv, at,
                                           preferred_element_type=jnp.float32)
        cur = _sum_k_smallest_rows(dall, 20)
        improved = jnp.logical_not(prev < cur)
        best_ref[...] = jnp.where(improved, v_ref[...], best_ref[...])
        return jnp.where(improved, cur, prev)

    jax.lax.fori_loop(0, epochs_ref[0], epoch_body, _INF)


def kernel(data, attract, repel, epochs):
    n, d = data.shape
    epochs_arr = jnp.asarray(epochs, jnp.int32).reshape(1)
    cat = jnp.concatenate([data.T, attract.T, repel.T], axis=1)
    return pl.pallas_call(
        _ng_kernel,
        out_shape=jax.ShapeDtypeStruct((n, d), jnp.float32),
        in_specs=[
            pl.BlockSpec(memory_space=pltpu.VMEM),
            pl.BlockSpec(memory_space=pltpu.VMEM),
            pl.BlockSpec(memory_space=pltpu.VMEM),
            pl.BlockSpec(memory_space=pltpu.VMEM),
            pl.BlockSpec(memory_space=pltpu.SMEM),
        ],
        out_specs=pl.BlockSpec(memory_space=pltpu.VMEM),
        scratch_shapes=[
            pltpu.VMEM((n, d), jnp.float32),
            pltpu.VMEM((n, n), jnp.float32),
        ],
    )(data, cat, attract, repel, epochs_arr)


# speedup vs baseline: 2.2979x; 1.5905x over previous
import jax
import jax.numpy as jnp
from jax.experimental import pallas as pl
from jax.experimental.pallas import tpu as pltpu

_A_O = 0.1
_K_O = 0.1
_A_ON = 0.006
_K_ON = 0.006
_A_R = 0.1
_K_R = 0.1
_A_RN = 0.006
_K_RN = 0.006
_MAXN = 16

_BIG_I = 2 ** 30
_INF = float("inf")


def _rayleigh(dx, k, a):
    return k / a * dx * jnp.exp(-0.5 * (dx / a) ** 2)


def _negexp(dx, k, a):
    return k * jnp.exp(-a * dx ** 2)


def _mask_k_smallest(score, k, rowwise=False):
    work = score
    for _ in range(k):
        m = jnp.min(work, axis=1, keepdims=True)
        if not rowwise:
            m = jnp.min(m, axis=0, keepdims=True)
        work = jnp.where(work == m, _INF, work)
    return work != score


def _cumsum_lanes(x):
    lane = jax.lax.broadcasted_iota(jnp.int32, x.shape, 1)
    y = x
    shift = 1
    while shift < x.shape[1]:
        y = y + jnp.where(lane >= shift, jnp.roll(y, shift, axis=1), 0.0)
        shift *= 2
    return y


def _sum_k_smallest_rows(score, k):
    work = score
    acc = jnp.zeros((score.shape[0], 1), jnp.float32)
    for _ in range(k):
        m = jnp.min(work, axis=1, keepdims=True)
        work = jnp.where(work == m, _INF, work)
        acc = acc + m
    return jnp.sum(acc)


def _ng_kernel(data_ref, cat_ref, attract_ref, repel_ref, epochs_ref,
               best_ref, v_ref, adj_ref):
    n, d = data_ref.shape
    na = attract_ref.shape[0]
    nr = repel_ref.shape[0]

    v_ref[...] = data_ref[...]
    best_ref[...] = data_ref[...]
    adj_ref[...] = jnp.zeros((n, n), jnp.float32)

    sq_cat = jnp.sum(cat_ref[...] ** 2, axis=0, keepdims=True)
    sq_d = sq_cat[:, :n]
    sq_a = sq_cat[:, n:n + na]
    sq_r = sq_cat[:, n + na:]

    lane_n = jax.lax.broadcasted_iota(jnp.int32, (1, n), 1)
    lane_a = jax.lax.broadcasted_iota(jnp.int32, (1, na), 1)
    slot_iota = jax.lax.broadcasted_iota(
        jnp.int32, (_MAXN, 1), 0).astype(jnp.float32)

    def node_step(s1, carry):
        v1 = v_ref[pl.ds(s1, 1), :]
        sq_v1 = jnp.sum(v1 * v1, axis=1, keepdims=True)
        prod = jnp.dot(v1, cat_ref[...],
                       preferred_element_type=jnp.float32)

        ds_sc = sq_d - 2.0 * prod[:, :n]
        work = ds_sc
        nbr = []
        hit0 = None
        for j in range(6):
            m = jnp.min(work, axis=1, keepdims=True)
            if j == 0:
                hit0 = ds_sc == m
            else:
                nbr.append(jnp.min(jnp.where(work == m, lane_n, _BIG_I)))
            work = jnp.where(work == m, _INF, work)
        mask5 = jnp.logical_and(work != ds_sc, jnp.logical_not(hit0))

        row = adj_ref[pl.ds(s1, 1), :]
        rowv = jnp.where(mask5, 1.0, row)
        adj_ref[pl.ds(s1, 1), :] = rowv
        for idx in nbr:
            r2 = adj_ref[pl.ds(idx, 1), :]
            adj_ref[pl.ds(idx, 1), :] = jnp.where(lane_n == s1, 1.0, r2)

        da = sq_a - 2.0 * prod[:, n:n + na]
        work = da
        gidx = []
        for j in range(3):
            m = jnp.min(work, axis=1, keepdims=True)
            gidx.append(jnp.min(jnp.where(work == m, lane_a, _BIG_I)))
            work = jnp.where(work == m, _INF, work)
        sel_a = work != da
        w_a = jnp.where(sel_a, _rayleigh(da + sq_v1, _A_O, _K_O), 0.0)
        move = (jnp.dot(w_a, attract_ref[...],
                        preferred_element_type=jnp.float32)
                - jnp.sum(w_a, axis=1, keepdims=True) * v1)
        gks = [attract_ref[pl.ds(i, 1), :] for i in gidx]

        dxr = sq_r + sq_v1 - 2.0 * prod[:, n + na:]
        sel_r = _mask_k_smallest(dxr, 10)
        w_r = jnp.where(sel_r, _negexp(dxr, _A_R, _K_R), 0.0)
        move = move - (jnp.dot(w_r, repel_ref[...],
                               preferred_element_type=jnp.float32)
                       - jnp.sum(w_r, axis=1, keepdims=True) * v1)
        v_ref[pl.ds(s1, 1), :] = v1 + move

        cnt = jnp.sum(rowv, axis=1, keepdims=True)
        p_set = _cumsum_lanes(rowv)
        p_unset = _cumsum_lanes(1.0 - rowv)
        slotid = jnp.where(rowv > 0.0, p_set - 1.0, cnt + p_unset - 1.0)
        onehot = jnp.where(slotid == slot_iota, 1.0, 0.0)
        valid = slot_iota < cnt

        vk = jnp.dot(onehot, v_ref[...],
                     preferred_element_type=jnp.float32)
        sq_vk = jnp.sum(vk * vk, axis=1, keepdims=True)

        mk = jnp.zeros((_MAXN, d), jnp.float32)
        for g in gks:
            diffn = g - vk
            dxn = jnp.sum(diffn * diffn, axis=1, keepdims=True)
            mk = mk + _rayleigh(dxn, _A_ON, _K_ON) * diffn

        rt = cat_ref[:, n + na:]
        drn = sq_vk + sq_r - 2.0 * jnp.dot(vk, rt,
                                           preferred_element_type=jnp.float32)
        sel_n = _mask_k_smallest(drn, 10, rowwise=True)
        w_n = jnp.where(sel_n, _negexp(drn, _A_RN, _K_RN), 0.0)
        w_sum = jnp.sum(w_n, axis=1, keepdims=True)
        mk = mk - (jnp.dot(w_n, repel_ref[...],
                           preferred_element_type=jnp.float32) - w_sum * vk)

        delta = jnp.where(valid, mk, 0.0)
        v_ref[...] = v_ref[...] + jax.lax.dot_general(
            onehot, delta, (((0,), (0,)), ((), ())),
            preferred_element_type=jnp.float32)
        return carry

    def epoch_body(e, prev):
        jax.lax.fori_loop(0, n, node_step, 0)
        vv = v_ref[...]
        sq_v = jnp.sum(vv * vv, axis=1, keepdims=True)
        at = cat_ref[:, n:n + na]
        dall = sq_v + sq_a - 2.0 * jnp.dot(vv, at,
                                           preferred_element_type=jnp.float32)
        cur = _sum_k_smallest_rows(dall, 20)
        improved = jnp.logical_not(prev < cur)
        best_ref[...] = jnp.where(improved, v_ref[...], best_ref[...])
        return jnp.where(improved, cur, prev)

    jax.lax.fori_loop(0, epochs_ref[0], epoch_body, _INF)


def kernel(data, attract, repel, epochs):
    n, d = data.shape
    epochs_arr = jnp.asarray(epochs, jnp.int32).reshape(1)
    cat = jnp.concatenate([data.T, attract.T, repel.T], axis=1)
    return pl.pallas_call(
        _ng_kernel,
        out_shape=jax.ShapeDtypeStruct((n, d), jnp.float32),
        in_specs=[
            pl.BlockSpec(memory_space=pltpu.VMEM),
            pl.BlockSpec(memory_space=pltpu.VMEM),
            pl.BlockSpec(memory_space=pltpu.VMEM),
            pl.BlockSpec(memory_space=pltpu.VMEM),
            pl.BlockSpec(memory_space=pltpu.SMEM),
        ],
        out_specs=pl.BlockSpec(memory_space=pltpu.VMEM),
        scratch_shapes=[
            pltpu.VMEM((n, d), jnp.float32),
            pltpu.VMEM((n, n), jnp.float32),
        ],
    )(data, cat, attract, repel, epochs_arr)


# speedup vs baseline: 2.5852x; 1.1250x over previous
import jax
import jax.numpy as jnp
from jax.experimental import pallas as pl
from jax.experimental.pallas import tpu as pltpu

_A_O = 0.1
_K_O = 0.1
_A_ON = 0.006
_K_ON = 0.006
_A_R = 0.1
_K_R = 0.1
_A_RN = 0.006
_K_RN = 0.006
_MAXN = 16

_BIG_I = 2 ** 30
_INF = float("inf")


def _rayleigh(dx, k, a):
    return k / a * dx * jnp.exp(-0.5 * (dx / a) ** 2)


def _negexp(dx, k, a):
    return k * jnp.exp(-a * dx ** 2)


def _mask_k_smallest(score, k, rowwise=False):
    work = score
    for _ in range(k):
        m = jnp.min(work, axis=1, keepdims=True)
        if not rowwise:
            m = jnp.min(m, axis=0, keepdims=True)
        work = jnp.where(work == m, _INF, work)
    return work != score


def _cumsum_lanes(x):
    lane = jax.lax.broadcasted_iota(jnp.int32, x.shape, 1)
    y = x
    shift = 1
    while shift < x.shape[1]:
        y = y + jnp.where(lane >= shift, jnp.roll(y, shift, axis=1), 0.0)
        shift *= 2
    return y


def _sum_k_smallest_rows(score, k):
    work = score
    acc = jnp.zeros((score.shape[0], 1), jnp.float32)
    for _ in range(k):
        m = jnp.min(work, axis=1, keepdims=True)
        work = jnp.where(work == m, _INF, work)
        acc = acc + m
    return jnp.sum(acc)


def _ng_kernel(data_ref, cat_ref, attract_ref, repel_ref, epochs_ref,
               best_ref, v_ref, adj_ref):
    n, d = data_ref.shape
    na = attract_ref.shape[0]
    nr = repel_ref.shape[0]

    v_ref[...] = data_ref[...]
    best_ref[...] = data_ref[...]
    adj_ref[...] = jnp.zeros((n, n), jnp.float32)

    sq_cat = jnp.sum(cat_ref[...] ** 2, axis=0, keepdims=True)
    sq_d = sq_cat[:, :n]
    sq_a = sq_cat[:, n:n + na]
    sq_r = sq_cat[:, n + na:]

    lane_n = jax.lax.broadcasted_iota(jnp.int32, (1, n), 1)
    lane_a = jax.lax.broadcasted_iota(jnp.int32, (1, na), 1)
    slot_iota = jax.lax.broadcasted_iota(
        jnp.int32, (_MAXN, 1), 0).astype(jnp.float32)

    def node_step(s1, carry):
        v1 = v_ref[pl.ds(s1, 1), :]
        sq_v1 = jnp.sum(v1 * v1, axis=1, keepdims=True)
        prod = jnp.dot(v1, cat_ref[...],
                       preferred_element_type=jnp.float32)

        ds_sc = sq_d - 2.0 * prod[:, :n]
        work = ds_sc
        nbr = []
        hit0 = None
        for j in range(6):
            m = jnp.min(work, axis=1, keepdims=True)
            if j == 0:
                hit0 = ds_sc == m
            else:
                nbr.append(jnp.min(jnp.where(work == m, lane_n, _BIG_I)))
            work = jnp.where(work == m, _INF, work)
        mask5 = jnp.logical_and(work != ds_sc, jnp.logical_not(hit0))

        row = adj_ref[pl.ds(s1, 1), :]
        rowv = jnp.where(mask5, 1.0, row)
        adj_ref[pl.ds(s1, 1), :] = rowv
        for idx in nbr:
            r2 = adj_ref[pl.ds(idx, 1), :]
            adj_ref[pl.ds(idx, 1), :] = jnp.where(lane_n == s1, 1.0, r2)

        da = sq_a - 2.0 * prod[:, n:n + na]
        work = da
        gidx = []
        for j in range(3):
            m = jnp.min(work, axis=1, keepdims=True)
            gidx.append(jnp.min(jnp.where(work == m, lane_a, _BIG_I)))
            work = jnp.where(work == m, _INF, work)
        sel_a = work != da
        w_a = jnp.where(sel_a, _rayleigh(da + sq_v1, _A_O, _K_O), 0.0)
        move = (jnp.dot(w_a, attract_ref[...],
                        preferred_element_type=jnp.float32)
                - jnp.sum(w_a, axis=1, keepdims=True) * v1)
        gks = [attract_ref[pl.ds(i, 1), :] for i in gidx]

        dxr = sq_r + sq_v1 - 2.0 * prod[:, n + na:]
        sel_r = _mask_k_smallest(dxr, 10)
        w_r = jnp.where(sel_r, _negexp(dxr, _A_R, _K_R), 0.0)
        move = move - (jnp.dot(w_r, repel_ref[...],
                               preferred_element_type=jnp.float32)
                       - jnp.sum(w_r, axis=1, keepdims=True) * v1)
        v_ref[pl.ds(s1, 1), :] = v1 + move

        cnt = jnp.sum(rowv, axis=1, keepdims=True)
        p_set = _cumsum_lanes(rowv)
        p_unset = _cumsum_lanes(1.0 - rowv)
        slotid = jnp.where(rowv > 0.0, p_set - 1.0, cnt + p_unset - 1.0)
        valid = slot_iota < cnt
        nidx = [jnp.min(jnp.where(slotid == float(j), lane_n, _BIG_I))
                for j in range(_MAXN)]

        vk = jnp.concatenate([v_ref[pl.ds(i, 1), :] for i in nidx],
                             axis=0)
        sq_vk = jnp.sum(vk * vk, axis=1, keepdims=True)

        mk = jnp.zeros((_MAXN, d), jnp.float32)
        for g in gks:
            diffn = g - vk
            dxn = jnp.sum(diffn * diffn, axis=1, keepdims=True)
            mk = mk + _rayleigh(dxn, _A_ON, _K_ON) * diffn

        rt = cat_ref[:, n + na:]
        drn = sq_vk + sq_r - 2.0 * jnp.dot(vk, rt,
                                           preferred_element_type=jnp.float32)
        sel_n = _mask_k_smallest(drn, 10, rowwise=True)
        w_n = jnp.where(sel_n, _negexp(drn, _A_RN, _K_RN), 0.0)
        w_sum = jnp.sum(w_n, axis=1, keepdims=True)
        mk = mk - (jnp.dot(w_n, repel_ref[...],
                           preferred_element_type=jnp.float32) - w_sum * vk)

        vk_new = jnp.where(valid, vk + mk, vk)
        for j in range(_MAXN):
            v_ref[pl.ds(nidx[j], 1), :] = vk_new[j:j + 1, :]
        return carry

    def epoch_body(e, prev):
        jax.lax.fori_loop(0, n, node_step, 0)
        vv = v_ref[...]
        sq_v = jnp.sum(vv * vv, axis=1, keepdims=True)
        at = cat_ref[:, n:n + na]
        dall = sq_v + sq_a - 2.0 * jnp.dot(vv, at,
                                           preferred_element_type=jnp.float32)
        cur = _sum_k_smallest_rows(dall, 20)
        improved = jnp.logical_not(prev < cur)
        best_ref[...] = jnp.where(improved, v_ref[...], best_ref[...])
        return jnp.where(improved, cur, prev)

    jax.lax.fori_loop(0, epochs_ref[0], epoch_body, _INF)


def kernel(data, attract, repel, epochs):
    n, d = data.shape
    epochs_arr = jnp.asarray(epochs, jnp.int32).reshape(1)
    cat = jnp.concatenate([data.T, attract.T, repel.T], axis=1)
    return pl.pallas_call(
        _ng_kernel,
        out_shape=jax.ShapeDtypeStruct((n, d), jnp.float32),
        in_specs=[
            pl.BlockSpec(memory_space=pltpu.VMEM),
            pl.BlockSpec(memory_space=pltpu.VMEM),
            pl.BlockSpec(memory_space=pltpu.VMEM),
            pl.BlockSpec(memory_space=pltpu.VMEM),
            pl.BlockSpec(memory_space=pltpu.SMEM),
        ],
        out_specs=pl.BlockSpec(memory_space=pltpu.VMEM),
        scratch_shapes=[
            pltpu.VMEM((n, d), jnp.float32),
            pltpu.VMEM((n, n), jnp.float32),
        ],
    )(data, cat, attract, repel, epochs_arr)
